# Initial kernel scaffold; baseline (speedup 1.0000x reference)
import jax
import jax.numpy as jnp
from jax.experimental import pallas as pl


def kernel(indices, table):
    raise NotImplementedError("write your pallas kernel here")


# speedup vs baseline: 1.3781x; 1.3781x over previous
import functools

import jax
import jax.numpy as jnp
from jax import lax
from jax.experimental import pallas as pl
from jax.experimental.pallas import tpu as pltpu
from jax.experimental.pallas import tpu_sc as plsc

_NC, _NS = 2, 16
_NW = _NC * _NS
_D = 1024
_TOTAL = 1024 * 50
_B_PER_W = _TOTAL // _NW
_CHUNK = 40
_N_CHUNKS = _B_PER_W // _CHUNK


def _make_gather():
    mesh = plsc.VectorSubcoreMesh(core_axis_name="c", subcore_axis_name="s")

    @functools.partial(
        pl.kernel,
        mesh=mesh,
        out_type=jax.ShapeDtypeStruct((_TOTAL, _D), jnp.float32),
        scratch_types=[
            pltpu.VMEM((_B_PER_W,), jnp.int32),
            pltpu.VMEM((_CHUNK, _D), jnp.float32),
            pltpu.SemaphoreType.DMA,
        ],
    )
    def gather_rows(table_hbm, idx_hbm, out_hbm, idx_v, buf, sem):
        wid = lax.axis_index("s") * _NC + lax.axis_index("c")
        base = wid * _B_PER_W
        pltpu.sync_copy(idx_hbm.at[pl.ds(base, _B_PER_W)], idx_v)

        def body(g, carry):
            off = g * _CHUNK
            pltpu.async_copy(
                table_hbm.at[idx_v.at[pl.ds(off, _CHUNK)]], buf, sem
            ).wait()
            pltpu.sync_copy(buf, out_hbm.at[pl.ds(base + off, _CHUNK)])
            return carry

        lax.fori_loop(0, _N_CHUNKS, body, 0)

    return gather_rows


_gather = _make_gather()


def kernel(indices, table):
    flat = indices.reshape(-1)
    out = _gather(table, flat)
    return out.reshape(indices.shape[0], indices.shape[1], _D)


# speedup vs baseline: 1.4369x; 1.0426x over previous
import functools

import jax
import jax.numpy as jnp
from jax import lax
from jax.experimental import pallas as pl
from jax.experimental.pallas import tpu as pltpu
from jax.experimental.pallas import tpu_sc as plsc

_NC, _NS = 2, 16
_NW = _NC * _NS
_D = 1024
_TOTAL = 1024 * 50
_B_PER_W = _TOTAL // _NW
_CHUNK = 40
_N_CHUNKS = _B_PER_W // _CHUNK


def _make_gather():
    mesh = plsc.VectorSubcoreMesh(core_axis_name="c", subcore_axis_name="s")

    @functools.partial(
        pl.kernel,
        mesh=mesh,
        out_type=jax.ShapeDtypeStruct((_TOTAL, _D), jnp.float32),
        scratch_types=[
            pltpu.VMEM((_B_PER_W,), jnp.int32),
            pltpu.VMEM((_CHUNK, _D), jnp.float32),
            pltpu.VMEM((_CHUNK, _D), jnp.float32),
            pltpu.SemaphoreType.DMA,
            pltpu.SemaphoreType.DMA,
            pltpu.SemaphoreType.DMA,
            pltpu.SemaphoreType.DMA,
        ],
    )
    def gather_rows(table_hbm, idx_hbm, out_hbm,
                    idx_v, buf0, buf1, g0, g1, o0, o1):
        wid = lax.axis_index("s") * _NC + lax.axis_index("c")
        base = wid * _B_PER_W
        pltpu.sync_copy(idx_hbm.at[pl.ds(base, _B_PER_W)], idx_v)

        def gather(g, buf, sem):
            off = g * _CHUNK
            return pltpu.make_async_copy(
                table_hbm.at[idx_v.at[pl.ds(off, _CHUNK)]], buf, sem)

        def out_copy(g, buf, sem):
            return pltpu.make_async_copy(
                buf, out_hbm.at[pl.ds(base + g * _CHUNK, _CHUNK)], sem)

        gather(0, buf0, g0).start()

        def body(p, carry):
            ga = 2 * p
            gather(ga, buf0, g0).wait()
            out_copy(ga, buf0, o0).start()
            gather(ga + 1, buf1, g1).start()
            gather(ga + 1, buf1, g1).wait()
            out_copy(ga + 1, buf1, o1).start()
            out_copy(ga, buf0, o0).wait()

            @pl.when(p < _N_CHUNKS // 2 - 1)
            def _():
                gather(ga + 2, buf0, g0).start()

            out_copy(ga + 1, buf1, o1).wait()
            return carry

        lax.fori_loop(0, _N_CHUNKS // 2, body, 0)

    return gather_rows


_gather = _make_gather()


def kernel(indices, table):
    flat = indices.reshape(-1)
    out = _gather(table, flat)
    return out.reshape(indices.shape[0], indices.shape[1], _D)


# speedup vs baseline: 1.9671x; 1.3690x over previous
import functools

import jax
import jax.numpy as jnp
from jax import lax
from jax.experimental import pallas as pl
from jax.experimental.pallas import tpu as pltpu
from jax.experimental.pallas import tpu_sc as plsc

_NC, _NS = 2, 16
_NW = _NC * _NS
_D = 1024
_BATCH = 1024
_SEQ = 50
_B_PER_W = _BATCH // _NW


def _make_gather():
    mesh = plsc.VectorSubcoreMesh(core_axis_name="c", subcore_axis_name="s")

    @functools.partial(
        pl.kernel,
        mesh=mesh,
        out_type=jax.ShapeDtypeStruct((_BATCH, _SEQ, _D), jnp.float32),
        scratch_types=[
            pltpu.VMEM((_B_PER_W, _SEQ), jnp.int32),
            pltpu.VMEM((_SEQ, _D), jnp.float32),
            pltpu.VMEM((_SEQ, _D), jnp.float32),
            pltpu.SemaphoreType.DMA,
            pltpu.SemaphoreType.DMA,
            pltpu.SemaphoreType.DMA,
            pltpu.SemaphoreType.DMA,
        ],
    )
    def gather_rows(table_hbm, idx_hbm, out_hbm,
                    idx_v, buf0, buf1, g0, g1, o0, o1):
        wid = lax.axis_index("s") * _NC + lax.axis_index("c")
        base = wid * _B_PER_W
        pltpu.sync_copy(idx_hbm.at[pl.ds(base, _B_PER_W)], idx_v)

        def gather(g, buf, sem):
            return pltpu.make_async_copy(
                table_hbm.at[idx_v.at[g]], buf, sem)

        def out_copy(g, buf, sem):
            return pltpu.make_async_copy(
                buf, out_hbm.at[base + g], sem)

        gather(0, buf0, g0).start()

        def body(p, carry):
            ga = 2 * p
            gather(ga, buf0, g0).wait()
            out_copy(ga, buf0, o0).start()
            gather(ga + 1, buf1, g1).start()
            gather(ga + 1, buf1, g1).wait()
            out_copy(ga + 1, buf1, o1).start()
            out_copy(ga, buf0, o0).wait()

            @pl.when(p < _B_PER_W // 2 - 1)
            def _():
                gather(ga + 2, buf0, g0).start()

            out_copy(ga + 1, buf1, o1).wait()
            return carry

        lax.fori_loop(0, _B_PER_W // 2, body, 0)

    return gather_rows


_gather = _make_gather()


def kernel(indices, table):
    return _gather(table, indices)
